# Initial kernel scaffold; baseline (speedup 1.0000x reference)
import jax
import jax.numpy as jnp
from jax.experimental import pallas as pl


def kernel(image_features, text_features):
    raise NotImplementedError("write your pallas kernel here")


# speedup vs baseline: 1.3742x; 1.3742x over previous
import functools

import jax
import jax.numpy as jnp
from jax.experimental import pallas as pl
from jax.experimental.pallas import tpu as pltpu

_LANE = 128
_EPS = 1e-30


def _round_up(x, m):
    return ((x + m - 1) // m) * m


_CHUNK = 512


def _chunked_softmax_write(probs_ref, chunks, valid):
    stats = []
    for j, lg in enumerate(chunks):
        if valid < (j + 1) * _CHUNK:
            col = j * _CHUNK + jax.lax.broadcasted_iota(jnp.int32, lg.shape, 1)
            lg = jnp.where(col < valid, lg, -jnp.inf)
        m_j = jnp.max(lg, axis=1, keepdims=True)
        e_j = jnp.exp(lg - m_j)
        s_j = jnp.sum(e_j, axis=1, keepdims=True)
        stats.append((m_j, e_j, s_j))
    m = functools.reduce(jnp.maximum, [m_j for m_j, _, _ in stats])
    alphas = [jnp.exp(m_j - m) for m_j, _, _ in stats]
    s = sum(a * s_j for a, (_, _, s_j) in zip(alphas, stats))
    r = pl.reciprocal(s, approx=True)
    for j, (a, (_, e_j, _)) in enumerate(zip(alphas, stats)):
        probs_ref[:, j * _CHUNK:(j + 1) * _CHUNK] = e_j * (a * r)


def _img_pass_kernel(img_ref, txt_ref, probs_ref, inv_ref, img16_ref, *,
                     m_valid):
    img = img_ref[...]
    sq = jnp.sum(img * img, axis=1, keepdims=True)
    scale = 100.0 * jax.lax.rsqrt(jnp.maximum(sq, _EPS))
    img_n = (img * scale).astype(jnp.bfloat16)
    txt = txt_ref[...]
    chunks = [
        jax.lax.dot_general(
            img_n, txt[c:c + _CHUNK], (((1,), (1,)), ((), ())),
            preferred_element_type=jnp.float32)
        for c in range(0, txt.shape[0], _CHUNK)
    ]
    _chunked_softmax_write(probs_ref, chunks, m_valid)
    inv_ref[...] = scale.T
    img16_ref[...] = img.astype(jnp.bfloat16)


def _txt_pass_kernel(txt_ref, img16_ref, inv_ref, probs_ref, *, n_valid):
    txt = txt_ref[...]
    img16 = img16_ref[...]
    inv = inv_ref[...]
    chunks = [
        jax.lax.dot_general(
            txt, img16[c:c + _CHUNK], (((1,), (1,)), ((), ())),
            preferred_element_type=jnp.float32)
        * inv[:, c:c + _CHUNK]
        for c in range(0, img16.shape[0], _CHUNK)
    ]
    _chunked_softmax_write(probs_ref, chunks, n_valid)


def kernel(image_features, text_features):
    N, D = image_features.shape
    M, Dt = text_features.shape
    assert D == Dt

    Np, Mp, Dp = _round_up(N, _LANE), _round_up(M, _LANE), _round_up(D, _LANE)
    img = image_features.astype(jnp.float32)
    if (Np, Dp) != (N, D):
        img = jnp.pad(img, ((0, Np - N), (0, Dp - D)))
    txt16 = text_features.astype(jnp.bfloat16)
    if (Mp, Dp) != (M, D):
        txt16 = jnp.pad(txt16, ((0, Mp - M), (0, Dp - D)))

    def tile(rows):
        for t in (512, 256, _LANE):
            if rows % t == 0:
                return t
        return _LANE

    tn, tm = tile(Np), tile(Mp)

    cparams = pltpu.CompilerParams(
        dimension_semantics=("parallel",),
        vmem_limit_bytes=60 * 1024 * 1024,
    )

    probs_img, inv_img, img16 = pl.pallas_call(
        functools.partial(_img_pass_kernel, m_valid=M),
        out_shape=(jax.ShapeDtypeStruct((Np, Mp), jnp.float32),
                   jax.ShapeDtypeStruct((1, Np), jnp.float32),
                   jax.ShapeDtypeStruct((Np, Dp), jnp.bfloat16)),
        grid=(Np // tn,),
        in_specs=[pl.BlockSpec((tn, Dp), lambda i: (i, 0)),
                  pl.BlockSpec((Mp, Dp), lambda i: (0, 0))],
        out_specs=(pl.BlockSpec((tn, Mp), lambda i: (i, 0)),
                   pl.BlockSpec((1, tn), lambda i: (0, i)),
                   pl.BlockSpec((tn, Dp), lambda i: (i, 0))),
        compiler_params=cparams,
    )(img, txt16)

    probs_txt = pl.pallas_call(
        functools.partial(_txt_pass_kernel, n_valid=N),
        out_shape=jax.ShapeDtypeStruct((Mp, Np), jnp.float32),
        grid=(Mp // tm,),
        in_specs=[pl.BlockSpec((tm, Dp), lambda j: (j, 0)),
                  pl.BlockSpec((Np, Dp), lambda j: (0, 0)),
                  pl.BlockSpec((1, Np), lambda j: (0, 0))],
        out_specs=pl.BlockSpec((tm, Np), lambda j: (j, 0)),
        compiler_params=cparams,
    )(txt16, img16, inv_img)

    return probs_img[:N, :M], probs_txt[:M, :N]
